# baseline (speedup 1.0000x reference)
import functools

import jax
import jax.numpy as jnp
from jax import lax
from jax.experimental import pallas as pl
from jax.experimental.pallas import tpu as pltpu
from jax.experimental.pallas import tpu_sc as plsc

_BATCH = 4
_SEQ = 8192
_HIDDEN = 1024
_NC = 2
_NS = 16
_NW = _NC * _NS
_ROWS_PER_W = _SEQ // _NW
_CHUNK = 16
_NCHUNK = _ROWS_PER_W // _CHUNK
_NBUF = 7
_AHEAD = 5


@functools.partial(
    pl.kernel,
    mesh=plsc.VectorSubcoreMesh(core_axis_name="c", subcore_axis_name="s"),
    out_type=jax.ShapeDtypeStruct((_BATCH, _SEQ, _HIDDEN), jnp.float32),
    scratch_types=(
        [pltpu.VMEM((_CHUNK, _HIDDEN), jnp.float32)] * _NBUF
        + [pltpu.SemaphoreType.DMA] * (2 * _NBUF)
    ),
)
def _broadcast_table(table_hbm, out_hbm, *scratch):
    bufs = scratch[:_NBUF]
    lsems = scratch[_NBUF:2 * _NBUF]
    ssems = scratch[2 * _NBUF:]
    wid = lax.axis_index("s") * _NC + lax.axis_index("c")
    base = wid * _ROWS_PER_W

    def load(i):
        r0 = base + i * _CHUNK
        return pltpu.make_async_copy(
            table_hbm.at[pl.ds(r0, _CHUNK), :], bufs[i % _NBUF], lsems[i % _NBUF])

    def stores(i):
        r0 = base + i * _CHUNK
        return [
            pltpu.make_async_copy(
                bufs[i % _NBUF], out_hbm.at[b, pl.ds(r0, _CHUNK), :],
                ssems[i % _NBUF])
            for b in range(_BATCH)
        ]

    for i in range(_AHEAD):
        load(i).start()
    for i in range(_NCHUNK):
        load(i).wait()
        for s in stores(i):
            s.start()
        j = i + _AHEAD
        if j < _NCHUNK:
            k = j - _NBUF
            if k >= 0:
                for s in stores(k):
                    s.wait()
            load(j).start()
    for i in range(_NCHUNK - _NBUF, _NCHUNK):
        for s in stores(i):
            s.wait()


def kernel(x, table):
    del x
    return _broadcast_table(table)


# speedup vs baseline: 1.0393x; 1.0393x over previous
import functools

import jax
import jax.numpy as jnp
from jax import lax
from jax.experimental import pallas as pl
from jax.experimental.pallas import tpu as pltpu
from jax.experimental.pallas import tpu_sc as plsc

_BATCH = 4
_SEQ = 8192
_HIDDEN = 1024
_NC = 2
_NS = 16
_NW = _NC * _NS
_ROWS_PER_W = _SEQ // _NW
_BUF_ROWS = 56
_SIZES = (56, 56, 56, 56, 32)
_OFFS = (0, 56, 112, 168, 224)
_NCHUNK = len(_SIZES)


@functools.partial(
    pl.kernel,
    mesh=plsc.VectorSubcoreMesh(core_axis_name="c", subcore_axis_name="s"),
    out_type=jax.ShapeDtypeStruct((_BATCH, _SEQ, _HIDDEN), jnp.float32),
    scratch_types=[
        pltpu.VMEM((_BUF_ROWS, _HIDDEN), jnp.float32),
        pltpu.VMEM((_BUF_ROWS, _HIDDEN), jnp.float32),
        pltpu.SemaphoreType.DMA,
        pltpu.SemaphoreType.DMA,
        pltpu.SemaphoreType.DMA,
        pltpu.SemaphoreType.DMA,
    ],
)
def _broadcast_table(table_hbm, out_hbm, buf0, buf1, lsem0, lsem1, ssem0, ssem1):
    wid = lax.axis_index("s") * _NC + lax.axis_index("c")
    base = wid * _ROWS_PER_W
    bufs = (buf0, buf1)
    lsems = (lsem0, lsem1)
    ssems = (ssem0, ssem1)

    def load(i):
        r0 = base + _OFFS[i]
        return pltpu.make_async_copy(
            table_hbm.at[pl.ds(r0, _SIZES[i]), :],
            bufs[i % 2].at[pl.ds(0, _SIZES[i]), :],
            lsems[i % 2])

    def stores(i):
        r0 = base + _OFFS[i]
        return [
            pltpu.make_async_copy(
                bufs[i % 2].at[pl.ds(0, _SIZES[i]), :],
                out_hbm.at[b, pl.ds(r0, _SIZES[i]), :],
                ssems[i % 2])
            for b in range(_BATCH)
        ]

    load(0).start()
    for i in range(_NCHUNK):
        load(i).wait()
        for s in stores(i):
            s.start()
        if i + 1 < _NCHUNK:
            if i >= 1:
                for s in stores(i - 1):
                    s.wait()
            load(i + 1).start()
    for i in (_NCHUNK - 2, _NCHUNK - 1):
        for s in stores(i):
            s.wait()


def kernel(x, table):
    del x
    return _broadcast_table(table)


# speedup vs baseline: 1.0465x; 1.0069x over previous
import functools

import jax
import jax.numpy as jnp
from jax import lax
from jax.experimental import pallas as pl
from jax.experimental.pallas import tpu as pltpu
from jax.experimental.pallas import tpu_sc as plsc

_BATCH = 4
_SEQ = 8192
_HIDDEN = 1024
_NC = 2
_NS = 16
_NW = _NC * _NS
_ROWS_PER_W = _SEQ // _NW
_SIZES = (64, 56, 64, 56, 16)
_OFFS = (0, 64, 120, 184, 240)
_NCHUNK = len(_SIZES)


@functools.partial(
    pl.kernel,
    mesh=plsc.VectorSubcoreMesh(core_axis_name="c", subcore_axis_name="s"),
    out_type=jax.ShapeDtypeStruct((_BATCH, _SEQ, _HIDDEN), jnp.float32),
    scratch_types=[
        pltpu.VMEM((64, _HIDDEN), jnp.float32),
        pltpu.VMEM((56, _HIDDEN), jnp.float32),
        pltpu.SemaphoreType.DMA,
        pltpu.SemaphoreType.DMA,
        pltpu.SemaphoreType.DMA,
        pltpu.SemaphoreType.DMA,
    ],
)
def _broadcast_table(table_hbm, out_hbm, buf0, buf1, lsem0, lsem1, ssem0, ssem1):
    wid = lax.axis_index("s") * _NC + lax.axis_index("c")
    base = wid * _ROWS_PER_W
    bufs = (buf0, buf1)
    lsems = (lsem0, lsem1)
    ssems = (ssem0, ssem1)

    def load(i):
        r0 = base + _OFFS[i]
        return pltpu.make_async_copy(
            table_hbm.at[pl.ds(r0, _SIZES[i]), :],
            bufs[i % 2].at[pl.ds(0, _SIZES[i]), :],
            lsems[i % 2])

    def stores(i):
        r0 = base + _OFFS[i]
        return [
            pltpu.make_async_copy(
                bufs[i % 2].at[pl.ds(0, _SIZES[i]), :],
                out_hbm.at[b, pl.ds(r0, _SIZES[i]), :],
                ssems[i % 2])
            for b in range(_BATCH)
        ]

    load(0).start()
    for i in range(_NCHUNK):
        load(i).wait()
        for s in stores(i):
            s.start()
        if i + 1 < _NCHUNK:
            if i >= 1:
                for s in stores(i - 1):
                    s.wait()
            load(i + 1).start()
    for i in (_NCHUNK - 2, _NCHUNK - 1):
        for s in stores(i):
            s.wait()


def kernel(x, table):
    del x
    return _broadcast_table(table)
